# Initial kernel scaffold; baseline (speedup 1.0000x reference)
import jax
import jax.numpy as jnp
from jax.experimental import pallas as pl


def kernel(x, edge_index, W1, b1, W2, b2):
    raise NotImplementedError("write your pallas kernel here")


# speedup vs baseline: 40.7656x; 40.7656x over previous
import functools

import jax
import jax.numpy as jnp
from jax import lax
from jax.experimental import pallas as pl
from jax.experimental.pallas import tpu as pltpu
from jax.experimental.pallas import tpu_sc as plsc

N = 50000
E = 1600000
DIN = 1433
DH = 16
DOUT = 7

B = 128
KI = 8
RW = 392
G = RW // KI
R_PAD = 32 * RW
E_PAD = R_PAD * B
N_PAD = 51200
NSUB = N_PAD // 16
BM = 1000
NBLK = N // BM

_mesh = plsc.VectorSubcoreMesh(
    core_axis_name="c", subcore_axis_name="s", num_cores=2, num_subcores=16)


@functools.partial(
    pl.kernel,
    out_type=jax.ShapeDtypeStruct((2, 1, N_PAD), jnp.float32),
    mesh=_mesh,
    scratch_types=[
        pltpu.VMEM((KI, B), jnp.int32),
        pltpu.VMEM((B,), jnp.float32),
        pltpu.VMEM_SHARED((N_PAD,), jnp.float32),
    ],
)
def _deg_kernel(dst_hbm, z1_hbm, ones_hbm, out_hbm, didx, ones_v, deg_s):
    cid = lax.axis_index("c")
    sid = lax.axis_index("s")
    wid = sid * 2 + cid
    pltpu.sync_copy(ones_hbm, ones_v)
    pltpu.sync_copy(z1_hbm.at[pl.ds(sid * NSUB, NSUB)],
                    deg_s.at[pl.ds(sid * NSUB, NSUB)])
    plsc.subcore_barrier()

    def body(g, carry):
        r0 = wid * RW + g * KI
        pltpu.sync_copy(dst_hbm.at[pl.ds(r0, KI)], didx)
        for j in range(KI):
            pltpu.sync_copy(ones_v, deg_s.at[didx.at[j]], add=True)
        return carry

    lax.fori_loop(0, G, body, 0)
    plsc.subcore_barrier()
    pltpu.sync_copy(deg_s.at[pl.ds(sid * NSUB, NSUB)],
                    out_hbm.at[cid, 0, pl.ds(sid * NSUB, NSUB)])


@functools.partial(
    pl.kernel,
    out_type=jax.ShapeDtypeStruct((2, N_PAD, DH), jnp.float32),
    mesh=_mesh,
    scratch_types=[
        pltpu.VMEM((KI, B), jnp.int32),
        pltpu.VMEM((KI, B), jnp.int32),
        pltpu.VMEM((KI, B, DH), jnp.float32),
        pltpu.VMEM_SHARED((N_PAD, DH), jnp.float32),
        pltpu.SemaphoreType.DMA,
    ],
    compiler_params=pltpu.CompilerParams(use_tc_tiling_on_sc=False),
)
def _agg_kernel(src_hbm, dst_hbm, tab_hbm, z2_hbm, out_hbm,
                sidx, didx, rows, acc_s, sem):
    cid = lax.axis_index("c")
    sid = lax.axis_index("s")
    wid = sid * 2 + cid
    pltpu.sync_copy(z2_hbm.at[pl.ds(sid * NSUB, NSUB)],
                    acc_s.at[pl.ds(sid * NSUB, NSUB)])
    plsc.subcore_barrier()

    def body(g, carry):
        r0 = wid * RW + g * KI
        pltpu.sync_copy(src_hbm.at[pl.ds(r0, KI)], sidx)
        pltpu.sync_copy(dst_hbm.at[pl.ds(r0, KI)], didx)
        descs = [pltpu.async_copy(tab_hbm.at[sidx.at[j]], rows.at[j], sem)
                 for j in range(KI)]
        for d in descs:
            d.wait()
        for j in range(KI):
            pltpu.sync_copy(rows.at[j], acc_s.at[didx.at[j]], add=True)
        return carry

    lax.fori_loop(0, G, body, 0)
    plsc.subcore_barrier()
    pltpu.sync_copy(acc_s.at[pl.ds(sid * NSUB, NSUB)],
                    out_hbm.at[cid, pl.ds(sid * NSUB, NSUB)])


def _dinv(degp):
    BD = 1024

    def body(dp_ref, o_ref):
        deg = dp_ref[0] + dp_ref[1] + 1.0
        dinv = lax.rsqrt(jnp.maximum(deg, 1.0))
        col = jnp.transpose(dinv, (1, 0))
        o_ref[...] = jnp.broadcast_to(col, (BD, DH))

    return pl.pallas_call(
        body,
        grid=(N_PAD // BD,),
        in_specs=[pl.BlockSpec((2, 1, BD), lambda i: (0, 0, i))],
        out_specs=pl.BlockSpec((BD, DH), lambda i: (i, 0)),
        out_shape=jax.ShapeDtypeStruct((N_PAD, DH), jnp.float32),
    )(degp)


def _mm1(x, W1, dv):
    def body(x_ref, w_ref, dv_ref, o_ref):
        o_ref[...] = jnp.dot(x_ref[...], w_ref[...],
                             preferred_element_type=jnp.float32) * dv_ref[...]

    return pl.pallas_call(
        body,
        grid=(NBLK,),
        in_specs=[
            pl.BlockSpec((BM, DIN), lambda i: (i, 0)),
            pl.BlockSpec((DIN, DH), lambda i: (0, 0)),
            pl.BlockSpec((BM, DH), lambda i: (i, 0)),
        ],
        out_specs=pl.BlockSpec((BM, DH), lambda i: (i, 0)),
        out_shape=jax.ShapeDtypeStruct((N, DH), jnp.float32),
    )(x, W1, dv)


def _layer2(acc, hp, dv, W2p, b1r):
    def body(a_ref, hp_ref, dv_ref, w_ref, b_ref, o_ref):
        dinv = dv_ref[...]
        s = a_ref[0] + a_ref[1] + hp_ref[...]
        h1 = jnp.maximum(dinv * s + b_ref[...], 0.0)
        o_ref[...] = jnp.dot(h1, w_ref[...],
                             preferred_element_type=jnp.float32) * dinv

    return pl.pallas_call(
        body,
        grid=(NBLK,),
        in_specs=[
            pl.BlockSpec((2, BM, DH), lambda i: (0, i, 0)),
            pl.BlockSpec((BM, DH), lambda i: (i, 0)),
            pl.BlockSpec((BM, DH), lambda i: (i, 0)),
            pl.BlockSpec((DH, DH), lambda i: (0, 0)),
            pl.BlockSpec((1, DH), lambda i: (0, 0)),
        ],
        out_specs=pl.BlockSpec((BM, DH), lambda i: (i, 0)),
        out_shape=jax.ShapeDtypeStruct((N, DH), jnp.float32),
    )(acc, hp, dv, W2p, b1r)


def _final(acc2, hp2, dv, b2r):
    def body(a_ref, hp_ref, dv_ref, b_ref, o_ref):
        s = a_ref[0] + a_ref[1] + hp_ref[...]
        o_ref[...] = dv_ref[...] * s + b_ref[...]

    return pl.pallas_call(
        body,
        grid=(NBLK,),
        in_specs=[
            pl.BlockSpec((2, BM, DH), lambda i: (0, i, 0)),
            pl.BlockSpec((BM, DH), lambda i: (i, 0)),
            pl.BlockSpec((BM, DH), lambda i: (i, 0)),
            pl.BlockSpec((1, DH), lambda i: (0, 0)),
        ],
        out_specs=pl.BlockSpec((BM, DH), lambda i: (i, 0)),
        out_shape=jax.ShapeDtypeStruct((N, DH), jnp.float32),
    )(acc2, hp2, dv, b2r)


def kernel(x, edge_index, W1, b1, W2, b2):
    ei = edge_index.astype(jnp.int32)
    src2d = jnp.concatenate(
        [ei[0], jnp.zeros((E_PAD - E,), jnp.int32)]).reshape(R_PAD, B)
    dst2d = jnp.concatenate(
        [ei[1], jnp.full((E_PAD - E,), N, jnp.int32)]).reshape(R_PAD, B)

    z1 = jnp.zeros((N_PAD,), jnp.float32)
    z2 = jnp.zeros((N_PAD, DH), jnp.float32)
    ones_b = jnp.ones((B,), jnp.float32)
    W2p = jnp.zeros((DH, DH), jnp.float32).at[:, :DOUT].set(W2)
    b1r = b1.reshape(1, DH)
    b2r = jnp.zeros((1, DH), jnp.float32).at[0, :DOUT].set(b2)

    degp = _deg_kernel(dst2d, z1, ones_b)
    dv = _dinv(degp)
    hp = _mm1(x, W1, dv)
    acc1 = _agg_kernel(src2d, dst2d, hp, z2)
    hp2 = _layer2(acc1, hp, dv, W2p, b1r)
    acc2 = _agg_kernel(src2d, dst2d, hp2, z2)
    outp = _final(acc2, hp2, dv, b2r)
    return outp[:, :DOUT]


# speedup vs baseline: 42.2062x; 1.0353x over previous
import functools

import jax
import jax.numpy as jnp
from jax import lax
from jax.experimental import pallas as pl
from jax.experimental.pallas import tpu as pltpu
from jax.experimental.pallas import tpu_sc as plsc

N = 50000
E = 1600000
DIN = 1433
DH = 16
DOUT = 7

B = 128
KI = 8
RW = 392
G = RW // KI
R_PAD = 32 * RW
E_PAD = R_PAD * B
N_PAD = 51200
NSUB = N_PAD // 16
BM = 1000
NBLK = N // BM

_mesh = plsc.VectorSubcoreMesh(
    core_axis_name="c", subcore_axis_name="s", num_cores=2, num_subcores=16)


@functools.partial(
    pl.kernel,
    out_type=jax.ShapeDtypeStruct((2, 1, N_PAD), jnp.float32),
    mesh=_mesh,
    scratch_types=[
        pltpu.VMEM((KI, B), jnp.int32),
        pltpu.VMEM((B,), jnp.float32),
        pltpu.VMEM_SHARED((N_PAD,), jnp.float32),
        pltpu.SemaphoreType.DMA,
    ],
)
def _deg_kernel(dst_hbm, z1_hbm, ones_hbm, out_hbm, didx, ones_v, deg_s, sem):
    cid = lax.axis_index("c")
    sid = lax.axis_index("s")
    wid = sid * 2 + cid
    pltpu.sync_copy(ones_hbm, ones_v)
    pltpu.sync_copy(z1_hbm.at[pl.ds(sid * NSUB, NSUB)],
                    deg_s.at[pl.ds(sid * NSUB, NSUB)])
    plsc.subcore_barrier()

    def body(g, carry):
        r0 = wid * RW + g * KI
        pltpu.sync_copy(dst_hbm.at[pl.ds(r0, KI)], didx)
        descs = [pltpu.async_copy(ones_v, deg_s.at[didx.at[j]], sem, add=True)
                 for j in range(KI)]
        for d in descs:
            d.wait()
        return carry

    lax.fori_loop(0, G, body, 0)
    plsc.subcore_barrier()
    pltpu.sync_copy(deg_s.at[pl.ds(sid * NSUB, NSUB)],
                    out_hbm.at[cid, 0, pl.ds(sid * NSUB, NSUB)])


@functools.partial(
    pl.kernel,
    out_type=jax.ShapeDtypeStruct((2, N_PAD, DH), jnp.float32),
    mesh=_mesh,
    scratch_types=[
        pltpu.VMEM((KI, B), jnp.int32),
        pltpu.VMEM((KI, B), jnp.int32),
        pltpu.VMEM((KI, B, DH), jnp.float32),
        pltpu.VMEM_SHARED((N_PAD, DH), jnp.float32),
        pltpu.SemaphoreType.DMA,
        pltpu.SemaphoreType.DMA,
    ],
    compiler_params=pltpu.CompilerParams(use_tc_tiling_on_sc=False),
)
def _agg_kernel(src_hbm, dst_hbm, tab_hbm, z2_hbm, out_hbm,
                sidx, didx, rows, acc_s, sem, sem2):
    cid = lax.axis_index("c")
    sid = lax.axis_index("s")
    wid = sid * 2 + cid
    pltpu.sync_copy(z2_hbm.at[pl.ds(sid * NSUB, NSUB)],
                    acc_s.at[pl.ds(sid * NSUB, NSUB)])
    plsc.subcore_barrier()

    def body(g, carry):
        r0 = wid * RW + g * KI
        pltpu.sync_copy(src_hbm.at[pl.ds(r0, KI)], sidx)
        pltpu.sync_copy(dst_hbm.at[pl.ds(r0, KI)], didx)
        descs = [pltpu.async_copy(tab_hbm.at[sidx.at[j]], rows.at[j], sem)
                 for j in range(KI)]
        for d in descs:
            d.wait()
        descs2 = [pltpu.async_copy(rows.at[j], acc_s.at[didx.at[j]], sem2,
                                   add=True)
                  for j in range(KI)]
        for d in descs2:
            d.wait()
        return carry

    lax.fori_loop(0, G, body, 0)
    plsc.subcore_barrier()
    pltpu.sync_copy(acc_s.at[pl.ds(sid * NSUB, NSUB)],
                    out_hbm.at[cid, pl.ds(sid * NSUB, NSUB)])


def _dinv(degp):
    BD = 1024

    def body(dp_ref, o_ref):
        deg = dp_ref[0] + dp_ref[1] + 1.0
        dinv = lax.rsqrt(jnp.maximum(deg, 1.0))
        col = jnp.transpose(dinv, (1, 0))
        o_ref[...] = jnp.broadcast_to(col, (BD, DH))

    return pl.pallas_call(
        body,
        grid=(N_PAD // BD,),
        in_specs=[pl.BlockSpec((2, 1, BD), lambda i: (0, 0, i))],
        out_specs=pl.BlockSpec((BD, DH), lambda i: (i, 0)),
        out_shape=jax.ShapeDtypeStruct((N_PAD, DH), jnp.float32),
    )(degp)


def _mm1(x, W1, dv):
    def body(x_ref, w_ref, dv_ref, o_ref):
        o_ref[...] = jnp.dot(x_ref[...], w_ref[...],
                             preferred_element_type=jnp.float32) * dv_ref[...]

    return pl.pallas_call(
        body,
        grid=(NBLK,),
        in_specs=[
            pl.BlockSpec((BM, DIN), lambda i: (i, 0)),
            pl.BlockSpec((DIN, DH), lambda i: (0, 0)),
            pl.BlockSpec((BM, DH), lambda i: (i, 0)),
        ],
        out_specs=pl.BlockSpec((BM, DH), lambda i: (i, 0)),
        out_shape=jax.ShapeDtypeStruct((N, DH), jnp.float32),
    )(x, W1, dv)


def _layer2(acc, hp, dv, W2p, b1r):
    def body(a_ref, hp_ref, dv_ref, w_ref, b_ref, o_ref):
        dinv = dv_ref[...]
        s = a_ref[0] + a_ref[1] + hp_ref[...]
        h1 = jnp.maximum(dinv * s + b_ref[...], 0.0)
        o_ref[...] = jnp.dot(h1, w_ref[...],
                             preferred_element_type=jnp.float32) * dinv

    return pl.pallas_call(
        body,
        grid=(NBLK,),
        in_specs=[
            pl.BlockSpec((2, BM, DH), lambda i: (0, i, 0)),
            pl.BlockSpec((BM, DH), lambda i: (i, 0)),
            pl.BlockSpec((BM, DH), lambda i: (i, 0)),
            pl.BlockSpec((DH, DH), lambda i: (0, 0)),
            pl.BlockSpec((1, DH), lambda i: (0, 0)),
        ],
        out_specs=pl.BlockSpec((BM, DH), lambda i: (i, 0)),
        out_shape=jax.ShapeDtypeStruct((N, DH), jnp.float32),
    )(acc, hp, dv, W2p, b1r)


def _final(acc2, hp2, dv, b2r):
    def body(a_ref, hp_ref, dv_ref, b_ref, o_ref):
        s = a_ref[0] + a_ref[1] + hp_ref[...]
        o_ref[...] = dv_ref[...] * s + b_ref[...]

    return pl.pallas_call(
        body,
        grid=(NBLK,),
        in_specs=[
            pl.BlockSpec((2, BM, DH), lambda i: (0, i, 0)),
            pl.BlockSpec((BM, DH), lambda i: (i, 0)),
            pl.BlockSpec((BM, DH), lambda i: (i, 0)),
            pl.BlockSpec((1, DH), lambda i: (0, 0)),
        ],
        out_specs=pl.BlockSpec((BM, DH), lambda i: (i, 0)),
        out_shape=jax.ShapeDtypeStruct((N, DH), jnp.float32),
    )(acc2, hp2, dv, b2r)


def kernel(x, edge_index, W1, b1, W2, b2):
    ei = edge_index.astype(jnp.int32)
    src2d = jnp.concatenate(
        [ei[0], jnp.zeros((E_PAD - E,), jnp.int32)]).reshape(R_PAD, B)
    dst2d = jnp.concatenate(
        [ei[1], jnp.full((E_PAD - E,), N, jnp.int32)]).reshape(R_PAD, B)

    z1 = jnp.zeros((N_PAD,), jnp.float32)
    z2 = jnp.zeros((N_PAD, DH), jnp.float32)
    ones_b = jnp.ones((B,), jnp.float32)
    W2p = jnp.zeros((DH, DH), jnp.float32).at[:, :DOUT].set(W2)
    b1r = b1.reshape(1, DH)
    b2r = jnp.zeros((1, DH), jnp.float32).at[0, :DOUT].set(b2)

    degp = _deg_kernel(dst2d, z1, ones_b)
    dv = _dinv(degp)
    hp = _mm1(x, W1, dv)
    acc1 = _agg_kernel(src2d, dst2d, hp, z2)
    hp2 = _layer2(acc1, hp, dv, W2p, b1r)
    acc2 = _agg_kernel(src2d, dst2d, hp2, z2)
    outp = _final(acc2, hp2, dv, b2r)
    return outp[:, :DOUT]
